# Initial kernel scaffold; baseline (speedup 1.0000x reference)
import jax
import jax.numpy as jnp
from jax.experimental import pallas as pl


def kernel(feat, feat_old, targets):
    raise NotImplementedError("write your pallas kernel here")


# speedup vs baseline: 2.6634x; 2.6634x over previous
import jax
import jax.numpy as jnp
from jax.experimental import pallas as pl
from jax.experimental.pallas import tpu as pltpu

_B, _D = 4096, 512
_TEMP = 0.01
_SCALE = 1.0 / _TEMP
_BLK = 256


def _norm_kernel(x_ref, out_ref):
    x = x_ref[...]
    n = jnp.sqrt(jnp.sum(x * x, axis=1, keepdims=True))
    out_ref[...] = (x / jnp.maximum(n, 1e-12)).astype(jnp.bfloat16)


def _loss_kernel(feat_ref, fo_ref, out_ref):
    i = pl.program_id(0)
    x = feat_ref[...]
    n = jnp.sqrt(jnp.sum(x * x, axis=1, keepdims=True))
    fn = (x / jnp.maximum(n, 1e-12)).astype(jnp.bfloat16)
    mm = jax.lax.dot_general(
        fn, fo_ref[...], (((1,), (1,)), ((), ())),
        preferred_element_type=jnp.float32)
    s = mm * _SCALE
    row = jax.lax.broadcasted_iota(jnp.int32, (_BLK, _B), 0)
    col = jax.lax.broadcasted_iota(jnp.int32, (_BLK, _B), 1)
    diag = col == row + i * _BLK
    pos = jnp.sum(jnp.where(diag, s, 0.0), axis=1)
    m = jnp.max(s, axis=1)
    sm = jnp.where(diag, -1e30, s)
    lse = m + jnp.log(
        jnp.sum(jnp.exp(sm - m[:, None]), axis=1) + jnp.exp(pos - m))
    part = jnp.sum(lse - pos).reshape(1, 1)

    @pl.when(i == 0)
    def _():
        out_ref[...] = jnp.zeros_like(out_ref)

    out_ref[...] += part


def kernel(feat, feat_old, targets):
    del targets
    fo_n = pl.pallas_call(
        _norm_kernel,
        out_shape=jax.ShapeDtypeStruct((_B, _D), jnp.bfloat16),
    )(feat_old)

    total = pl.pallas_call(
        _loss_kernel,
        grid=(_B // _BLK,),
        in_specs=[
            pl.BlockSpec((_BLK, _D), lambda i: (i, 0)),
            pl.BlockSpec((_B, _D), lambda i: (0, 0)),
        ],
        out_specs=pl.BlockSpec((1, 1), lambda i: (0, 0)),
        out_shape=jax.ShapeDtypeStruct((1, 1), jnp.float32),
        compiler_params=pltpu.CompilerParams(
            dimension_semantics=("arbitrary",)),
    )(feat, fo_n)

    return total[0, 0] * (1.0 / _B)


# speedup vs baseline: 2.7999x; 1.0513x over previous
import jax
import jax.numpy as jnp
from jax.experimental import pallas as pl
from jax.experimental.pallas import tpu as pltpu

_B, _D = 4096, 512
_TEMP = 0.01
_SCALE = 1.0 / _TEMP
_BLK = 256


def _norm_kernel(x_ref, out_ref):
    x = x_ref[...]
    n = jnp.sqrt(jnp.sum(x * x, axis=1, keepdims=True))
    out_ref[...] = (x / jnp.maximum(n, 1e-12)).astype(jnp.bfloat16)


def _loss_kernel(feat_ref, fo_ref, out_ref):
    i = pl.program_id(0)
    x = feat_ref[...]
    n = jnp.sqrt(jnp.sum(x * x, axis=1, keepdims=True))
    fn = x * (_SCALE / jnp.maximum(n, 1e-12))
    fnb = fn.astype(jnp.bfloat16)
    mm = jax.lax.dot_general(
        fnb, fo_ref[...], (((1,), (1,)), ((), ())),
        preferred_element_type=jnp.float32)
    pos = jnp.sum(fn * fo_ref[pl.ds(i * _BLK, _BLK), :].astype(jnp.float32),
                  axis=1)
    m = jnp.max(mm, axis=1)
    lse = m + jnp.log(jnp.sum(jnp.exp(mm - m[:, None]), axis=1))
    part = jnp.sum(lse - pos).reshape(1, 1)

    @pl.when(i == 0)
    def _():
        out_ref[...] = jnp.zeros_like(out_ref)

    out_ref[...] += part


def kernel(feat, feat_old, targets):
    del targets
    fo_n = pl.pallas_call(
        _norm_kernel,
        out_shape=jax.ShapeDtypeStruct((_B, _D), jnp.bfloat16),
    )(feat_old)

    total = pl.pallas_call(
        _loss_kernel,
        grid=(_B // _BLK,),
        in_specs=[
            pl.BlockSpec((_BLK, _D), lambda i: (i, 0)),
            pl.BlockSpec((_B, _D), lambda i: (0, 0)),
        ],
        out_specs=pl.BlockSpec((1, 1), lambda i: (0, 0)),
        out_shape=jax.ShapeDtypeStruct((1, 1), jnp.float32),
        compiler_params=pltpu.CompilerParams(
            dimension_semantics=("arbitrary",)),
    )(feat, fo_n)

    return total[0, 0] * (1.0 / _B)


# speedup vs baseline: 3.0563x; 1.0916x over previous
import jax
import jax.numpy as jnp
from jax.experimental import pallas as pl
from jax.experimental.pallas import tpu as pltpu

_B, _D = 4096, 512
_TEMP = 0.01
_SCALE = 1.0 / _TEMP
_BLK = 512


def _norm_kernel(x_ref, out_ref):
    x = x_ref[...]
    n = jnp.sqrt(jnp.sum(x * x, axis=1, keepdims=True))
    out_ref[...] = (x / jnp.maximum(n, 1e-12)).astype(jnp.bfloat16)


def _loss_kernel(feat_ref, fo_ref, out_ref):
    i = pl.program_id(0)
    x = feat_ref[...]
    n = jnp.sqrt(jnp.sum(x * x, axis=1, keepdims=True))
    fn = x * (_SCALE / jnp.maximum(n, 1e-12))
    fnb = fn.astype(jnp.bfloat16)
    mm = jax.lax.dot_general(
        fnb, fo_ref[...], (((1,), (1,)), ((), ())),
        preferred_element_type=jnp.float32)
    pos = jnp.sum(fn * fo_ref[pl.ds(i * _BLK, _BLK), :].astype(jnp.float32),
                  axis=1)
    m = jnp.max(mm, axis=1)
    lse = m + jnp.log(jnp.sum(jnp.exp(mm - m[:, None]), axis=1))
    part = jnp.sum(lse - pos).reshape(1, 1)

    @pl.when(i == 0)
    def _():
        out_ref[...] = jnp.zeros_like(out_ref)

    out_ref[...] += part


def kernel(feat, feat_old, targets):
    del targets
    fo_n = pl.pallas_call(
        _norm_kernel,
        out_shape=jax.ShapeDtypeStruct((_B, _D), jnp.bfloat16),
    )(feat_old)

    total = pl.pallas_call(
        _loss_kernel,
        grid=(_B // _BLK,),
        in_specs=[
            pl.BlockSpec((_BLK, _D), lambda i: (i, 0)),
            pl.BlockSpec((_B, _D), lambda i: (0, 0)),
        ],
        out_specs=pl.BlockSpec((1, 1), lambda i: (0, 0)),
        out_shape=jax.ShapeDtypeStruct((1, 1), jnp.float32),
        compiler_params=pltpu.CompilerParams(
            dimension_semantics=("arbitrary",)),
    )(feat, fo_n)

    return total[0, 0] * (1.0 / _B)


# speedup vs baseline: 3.2387x; 1.0597x over previous
import jax
import jax.numpy as jnp
from jax.experimental import pallas as pl
from jax.experimental.pallas import tpu as pltpu

_B, _D = 4096, 512
_TEMP = 0.01
_SCALE = 1.0 / _TEMP
_BLK = 512
_CHUNK = 1024


def _norm_kernel(x_ref, out_ref):
    x = x_ref[...]
    n = jnp.sqrt(jnp.sum(x * x, axis=1, keepdims=True))
    out_ref[...] = (x / jnp.maximum(n, 1e-12)).astype(jnp.bfloat16)


def _loss_kernel(feat_ref, fo_ref, out_ref):
    i = pl.program_id(0)
    x = feat_ref[...]
    n = jnp.sqrt(jnp.sum(x * x, axis=1, keepdims=True))
    fn = x * (_SCALE / jnp.maximum(n, 1e-12))
    fnb = fn.astype(jnp.bfloat16)
    pos = jnp.sum(fn * fo_ref[pl.ds(i * _BLK, _BLK), :].astype(jnp.float32),
                  axis=1)
    mks, sks = [], []
    for k in range(_B // _CHUNK):
        mm = jax.lax.dot_general(
            fnb, fo_ref[pl.ds(k * _CHUNK, _CHUNK), :],
            (((1,), (1,)), ((), ())),
            preferred_element_type=jnp.float32)
        mk = jnp.max(mm, axis=1)
        mks.append(mk)
        sks.append(jnp.sum(jnp.exp(mm - mk[:, None]), axis=1))
    m = mks[0]
    for mk in mks[1:]:
        m = jnp.maximum(m, mk)
    s = sks[0] * jnp.exp(mks[0] - m)
    for mk, sk in zip(mks[1:], sks[1:]):
        s += sk * jnp.exp(mk - m)
    lse = m + jnp.log(s)
    part = jnp.sum(lse - pos).reshape(1, 1)

    @pl.when(i == 0)
    def _():
        out_ref[...] = jnp.zeros_like(out_ref)

    out_ref[...] += part


def kernel(feat, feat_old, targets):
    del targets
    fo_n = pl.pallas_call(
        _norm_kernel,
        out_shape=jax.ShapeDtypeStruct((_B, _D), jnp.bfloat16),
    )(feat_old)

    total = pl.pallas_call(
        _loss_kernel,
        grid=(_B // _BLK,),
        in_specs=[
            pl.BlockSpec((_BLK, _D), lambda i: (i, 0)),
            pl.BlockSpec((_B, _D), lambda i: (0, 0)),
        ],
        out_specs=pl.BlockSpec((1, 1), lambda i: (0, 0)),
        out_shape=jax.ShapeDtypeStruct((1, 1), jnp.float32),
        compiler_params=pltpu.CompilerParams(
            dimension_semantics=("arbitrary",)),
    )(feat, fo_n)

    return total[0, 0] * (1.0 / _B)


# speedup vs baseline: 3.5911x; 1.1088x over previous
import jax
import jax.numpy as jnp
from jax.experimental import pallas as pl
from jax.experimental.pallas import tpu as pltpu

_B, _D = 4096, 512
_TEMP = 0.01
_SCALE = 1.0 / _TEMP
_BLK = 512
_CHUNK = 1024


def _loss_kernel(feat_ref, fo_ref, out_ref, fob_ref):
    i = pl.program_id(0)

    @pl.when(i == 0)
    def _():
        fo = fo_ref[...]
        no = jnp.sqrt(jnp.sum(fo * fo, axis=1, keepdims=True))
        fob_ref[...] = (fo / jnp.maximum(no, 1e-12)).astype(jnp.bfloat16)

    x = feat_ref[...]
    n = jnp.sqrt(jnp.sum(x * x, axis=1, keepdims=True))
    fn = x * (_SCALE / jnp.maximum(n, 1e-12))
    fnb = fn.astype(jnp.bfloat16)
    pos = jnp.sum(fn * fob_ref[pl.ds(i * _BLK, _BLK), :].astype(jnp.float32),
                  axis=1)
    mks, sks = [], []
    for k in range(_B // _CHUNK):
        mm = jax.lax.dot_general(
            fnb, fob_ref[pl.ds(k * _CHUNK, _CHUNK), :],
            (((1,), (1,)), ((), ())),
            preferred_element_type=jnp.float32)
        mk = jnp.max(mm, axis=1)
        mks.append(mk)
        sks.append(jnp.sum(jnp.exp(mm - mk[:, None]), axis=1))
    m = mks[0]
    for mk in mks[1:]:
        m = jnp.maximum(m, mk)
    s = sks[0] * jnp.exp(mks[0] - m)
    for mk, sk in zip(mks[1:], sks[1:]):
        s += sk * jnp.exp(mk - m)
    lse = m + jnp.log(s)
    part = jnp.sum(lse - pos).reshape(1, 1)

    @pl.when(i == 0)
    def _():
        out_ref[...] = jnp.zeros_like(out_ref)

    out_ref[...] += part


def kernel(feat, feat_old, targets):
    del targets
    total = pl.pallas_call(
        _loss_kernel,
        grid=(_B // _BLK,),
        in_specs=[
            pl.BlockSpec((_BLK, _D), lambda i: (i, 0)),
            pl.BlockSpec((_B, _D), lambda i: (0, 0)),
        ],
        out_specs=pl.BlockSpec((1, 1), lambda i: (0, 0)),
        out_shape=jax.ShapeDtypeStruct((1, 1), jnp.float32),
        scratch_shapes=[pltpu.VMEM((_B, _D), jnp.bfloat16)],
        compiler_params=pltpu.CompilerParams(
            dimension_semantics=("arbitrary",)),
    )(feat, feat_old)

    return total[0, 0] * (1.0 / _B)


# speedup vs baseline: 4.3074x; 1.1995x over previous
import jax
import jax.numpy as jnp
from jax.experimental import pallas as pl
from jax.experimental.pallas import tpu as pltpu

_B, _D = 4096, 512
_TEMP = 0.01
_SCALE = 1.0 / _TEMP
_BLK = 512
_CHUNK = 1024
_SHIFT = 24.0


def _loss_kernel(feat_ref, fo_ref, out_ref, fob_ref):
    i = pl.program_id(0)

    @pl.when(i == 0)
    def _():
        fo = fo_ref[...]
        no = jnp.sqrt(jnp.sum(fo * fo, axis=1, keepdims=True))
        fob_ref[...] = (fo / jnp.maximum(no, 1e-12)).astype(jnp.bfloat16)

    x = feat_ref[...]
    n = jnp.sqrt(jnp.sum(x * x, axis=1, keepdims=True))
    fn = x * (_SCALE / jnp.maximum(n, 1e-12))
    fnb = fn.astype(jnp.bfloat16)
    pos = jnp.sum(fn * fob_ref[pl.ds(i * _BLK, _BLK), :].astype(jnp.float32),
                  axis=1)
    sks = []
    for k in range(_B // _CHUNK):
        mm = jax.lax.dot_general(
            fnb, fob_ref[pl.ds(k * _CHUNK, _CHUNK), :],
            (((1,), (1,)), ((), ())),
            preferred_element_type=jnp.float32)
        sks.append(jnp.sum(jnp.exp(mm - _SHIFT), axis=1))
    s = sks[0]
    for sk in sks[1:]:
        s = s + sk
    lse = _SHIFT + jnp.log(s)
    part = jnp.sum(lse - pos).reshape(1, 1)

    @pl.when(i == 0)
    def _():
        out_ref[...] = jnp.zeros_like(out_ref)

    out_ref[...] += part


def kernel(feat, feat_old, targets):
    del targets
    total = pl.pallas_call(
        _loss_kernel,
        grid=(_B // _BLK,),
        in_specs=[
            pl.BlockSpec((_BLK, _D), lambda i: (i, 0)),
            pl.BlockSpec((_B, _D), lambda i: (0, 0)),
        ],
        out_specs=pl.BlockSpec((1, 1), lambda i: (0, 0)),
        out_shape=jax.ShapeDtypeStruct((1, 1), jnp.float32),
        scratch_shapes=[pltpu.VMEM((_B, _D), jnp.bfloat16)],
        compiler_params=pltpu.CompilerParams(
            dimension_semantics=("arbitrary",)),
    )(feat, feat_old)

    return total[0, 0] * (1.0 / _B)


# speedup vs baseline: 4.4433x; 1.0316x over previous
import jax
import jax.numpy as jnp
from jax.experimental import pallas as pl
from jax.experimental.pallas import tpu as pltpu

_B, _D = 4096, 512
_TEMP = 0.01
_SCALE = 1.0 / _TEMP
_BLK = 1024
_CHUNK = 1024
_SHIFT = 24.0


def _loss_kernel(feat_ref, fo_ref, out_ref, fob_ref):
    i = pl.program_id(0)

    @pl.when(i == 0)
    def _():
        fo = fo_ref[...]
        no = jnp.sqrt(jnp.sum(fo * fo, axis=1, keepdims=True))
        fob_ref[...] = (fo / jnp.maximum(no, 1e-12)).astype(jnp.bfloat16)

    x = feat_ref[...]
    n = jnp.sqrt(jnp.sum(x * x, axis=1, keepdims=True))
    fn = x * (_SCALE / jnp.maximum(n, 1e-12))
    fnb = fn.astype(jnp.bfloat16)
    pos = jnp.sum(fn * fob_ref[pl.ds(i * _BLK, _BLK), :].astype(jnp.float32),
                  axis=1)
    sks = []
    for k in range(_B // _CHUNK):
        mm = jax.lax.dot_general(
            fnb, fob_ref[pl.ds(k * _CHUNK, _CHUNK), :],
            (((1,), (1,)), ((), ())),
            preferred_element_type=jnp.float32)
        sks.append(jnp.sum(jnp.exp(mm - _SHIFT), axis=1))
    s = sks[0]
    for sk in sks[1:]:
        s = s + sk
    lse = _SHIFT + jnp.log(s)
    part = jnp.sum(lse - pos).reshape(1, 1)

    @pl.when(i == 0)
    def _():
        out_ref[...] = jnp.zeros_like(out_ref)

    out_ref[...] += part


def kernel(feat, feat_old, targets):
    del targets
    total = pl.pallas_call(
        _loss_kernel,
        grid=(_B // _BLK,),
        in_specs=[
            pl.BlockSpec((_BLK, _D), lambda i: (i, 0)),
            pl.BlockSpec((_B, _D), lambda i: (0, 0)),
        ],
        out_specs=pl.BlockSpec((1, 1), lambda i: (0, 0)),
        out_shape=jax.ShapeDtypeStruct((1, 1), jnp.float32),
        scratch_shapes=[pltpu.VMEM((_B, _D), jnp.bfloat16)],
        compiler_params=pltpu.CompilerParams(
            dimension_semantics=("arbitrary",)),
    )(feat, feat_old)

    return total[0, 0] * (1.0 / _B)
